# baseline (speedup 1.0000x reference)
import functools

import jax
import jax.numpy as jnp
from jax import lax
from jax.experimental import pallas as pl
from jax.experimental.pallas import tpu as pltpu
from jax.experimental.pallas import tpu_sc as plsc

D = 64
SCALE = 8.0
NC, NS = 2, 16
NW = NC * NS
CHUNK = 512
KSUB = CHUNK // 128


@jax.jit
def _lookup(table, idx2d):
    B = idx2d.shape[0] * 128
    b_per_w = B // NW
    n_chunks = b_per_w // CHUNK
    mesh = plsc.VectorSubcoreMesh(core_axis_name="c", subcore_axis_name="s")

    def body(table_hbm, idx_hbm, out_hbm, idx_v, rows_v, sem):
        wid = lax.axis_index("s") * NC + lax.axis_index("c")
        base_row = wid * (b_per_w // 128)

        def chunk(g, carry):
            r0 = base_row + g * KSUB
            pltpu.sync_copy(idx_hbm.at[pl.ds(r0, KSUB)], idx_v)
            cps = [
                pltpu.async_copy(
                    table_hbm.at[idx_v.at[j]],
                    rows_v.at[pl.ds(j * 128, 128)],
                    sem,
                )
                for j in range(KSUB)
            ]
            for c in cps:
                c.wait()

            def scale_row(r, c2):
                for c4 in range(4):
                    sl = pl.ds(c4 * 16, 16)
                    rows_v[r, sl] = rows_v[r, sl] * SCALE
                return c2

            lax.fori_loop(0, CHUNK, scale_row, 0)
            pltpu.sync_copy(
                rows_v, out_hbm.at[pl.ds(wid * b_per_w + g * CHUNK, CHUNK)]
            )
            return carry

        lax.fori_loop(0, n_chunks, chunk, 0)

    f = pl.kernel(
        body,
        out_type=jax.ShapeDtypeStruct((B, D), jnp.float32),
        mesh=mesh,
        compiler_params=pltpu.CompilerParams(use_tc_tiling_on_sc=False),
        scratch_types=[
            pltpu.VMEM((KSUB, 128), jnp.int32),
            pltpu.VMEM((CHUNK, D), jnp.float32),
            pltpu.SemaphoreType.DMA,
        ],
    )
    return f(table, idx2d)


def kernel(x, emb_weight):
    s0, s1 = x.shape
    B = s0 * s1
    idx2d = x.reshape(B // 128, 128).astype(jnp.int32)
    out = _lookup(emb_weight, idx2d)
    return out.reshape(s0, s1, D)
